# Initial kernel scaffold; baseline (speedup 1.0000x reference)
import jax
import jax.numpy as jnp
from jax.experimental import pallas as pl


def kernel(x, edge_index, W0, b0, W1, b1, W2, b2, g0, be0, g1, be1, g2, be2, Wo, bo):
    raise NotImplementedError("write your pallas kernel here")


# speedup vs baseline: 20.4726x; 20.4726x over previous
import functools

import jax
import jax.numpy as jnp
from jax import lax
from jax.experimental import pallas as pl
from jax.experimental.pallas import tpu as pltpu
from jax.experimental.pallas import tpu_sc as plsc

N = 10000
NP = 10240
E = 320000
HID = 64

NC = 2
NS = 16
NW = NC * NS
EPW = E // NW
CH = 80
NCHUNK = EPW // CH
RPT = NP // NS
ZROWS = 128


_sc_mesh = plsc.VectorSubcoreMesh(core_axis_name="c", subcore_axis_name="s")


@functools.partial(
    pl.kernel,
    mesh=_sc_mesh,
    compiler_params=pltpu.CompilerParams(use_tc_tiling_on_sc=False),
    out_type=jax.ShapeDtypeStruct((NC, NP, HID), jnp.float32),
    scratch_types=[
        pltpu.VMEM((NCHUNK, CH), jnp.int32),
        pltpu.VMEM((NCHUNK, CH), jnp.int32),
        pltpu.VMEM((CH, HID), jnp.float32),
        pltpu.VMEM((CH, HID), jnp.float32),
        pltpu.VMEM((ZROWS, HID), jnp.float32),
        pltpu.VMEM_SHARED((NP, HID), jnp.float32),
        pltpu.SemaphoreType.DMA,
        pltpu.SemaphoreType.DMA,
    ],
)
def _edge_scatter(y_hbm, src_hbm, dst_hbm, zout_hbm,
                  sidx, didx, rows0, rows1, zbuf, zsh, sem0, sem1):
    c = lax.axis_index("c")
    s = lax.axis_index("s")
    wid = c * NS + s

    pltpu.sync_copy(src_hbm.at[wid], sidx)
    pltpu.sync_copy(dst_hbm.at[wid], didx)

    def _zrow(r, carry):
        for q in range(HID // 16):
            zbuf[r, pl.ds(q * 16, 16)] = jnp.zeros((16,), jnp.float32)
        return carry
    lax.fori_loop(0, ZROWS, _zrow, 0)
    for k in range(RPT // ZROWS):
        pltpu.sync_copy(zbuf, zsh.at[pl.ds(s * RPT + k * ZROWS, ZROWS)])
    plsc.subcore_barrier()

    rows = (rows0, rows1)
    sems = (sem0, sem1)

    pltpu.async_copy(y_hbm.at[sidx.at[0]], rows0, sem0)

    def _body(k, carry):
        for b in range(2):
            j = 2 * k + b
            pltpu.make_async_copy(y_hbm.at[pl.ds(0, CH)], rows[b], sems[b]).wait()
            pltpu.async_copy(y_hbm.at[sidx.at[j + 1]], rows[1 - b], sems[1 - b])
            pltpu.sync_copy(rows[b], zsh.at[didx.at[j]], add=True)
        return carry
    lax.fori_loop(0, (NCHUNK - 1) // 2, _body, 0)

    pltpu.make_async_copy(y_hbm.at[pl.ds(0, CH)], rows0, sem0).wait()
    pltpu.sync_copy(rows0, zsh.at[didx.at[NCHUNK - 1]], add=True)

    plsc.subcore_barrier()

    for k in range(RPT // ZROWS):
        base = s * RPT + k * ZROWS
        pltpu.sync_copy(zsh.at[pl.ds(base, ZROWS)],
                        zout_hbm.at[c, pl.ds(base, ZROWS)])


def _row_mask():
    return lax.broadcasted_iota(jnp.int32, (NP, 1), 0) < N


def _tc0_body(x_ref, w0_ref, degc_ref, dinv_ref, y_ref):
    degc = degc_ref[...]
    deg = degc[0] + degc[1] + 1.0
    dinv = lax.rsqrt(deg)
    xw = jnp.dot(x_ref[...], w0_ref[...], preferred_element_type=jnp.float32)
    dinv_ref[...] = dinv
    y_ref[...] = xw * dinv


_tc0 = pl.pallas_call(
    _tc0_body,
    out_shape=(jax.ShapeDtypeStruct((NP, 1), jnp.float32),
               jax.ShapeDtypeStruct((NP, HID), jnp.float32)),
)


def _bn_relu(h, g, be):
    mask = _row_mask()
    hm = jnp.where(mask, h, 0.0)
    mean = jnp.sum(hm, axis=0, keepdims=True) / N
    dev = jnp.where(mask, h - mean, 0.0)
    var = jnp.sum(dev * dev, axis=0, keepdims=True) / N
    hn = (h - mean) * lax.rsqrt(var + 1e-5) * g + be
    return jnp.where(mask, jnp.maximum(hn, 0.0), 0.0)


def _tc_mid_body(zp_ref, y_ref, dinv_ref, b_ref, g_ref, be_ref, w_ref, yo_ref):
    zp = zp_ref[...]
    dinv = dinv_ref[...]
    h = (zp[0] + zp[1] + y_ref[...]) * dinv + b_ref[...]
    h = _bn_relu(h, g_ref[...], be_ref[...])
    yo_ref[...] = jnp.dot(h, w_ref[...], preferred_element_type=jnp.float32) * dinv


_tc_mid = pl.pallas_call(
    _tc_mid_body,
    out_shape=jax.ShapeDtypeStruct((NP, HID), jnp.float32),
)


def _tc_fin_body(zp_ref, y_ref, dinv_ref, b_ref, g_ref, be_ref,
                 wo_ref, bo_ref, o_ref):
    zp = zp_ref[...]
    h = (zp[0] + zp[1] + y_ref[...]) * dinv_ref[...] + b_ref[...]
    h = _bn_relu(h, g_ref[...], be_ref[...])
    o_ref[...] = jnp.dot(h, wo_ref[...], preferred_element_type=jnp.float32) \
        + bo_ref[...]


_tc_fin = pl.pallas_call(
    _tc_fin_body,
    out_shape=jax.ShapeDtypeStruct((NP, 1), jnp.float32),
)


def kernel(x, edge_index, W0, b0, W1, b1, W2, b2,
           g0, be0, g1, be1, g2, be2, Wo, bo):
    ei = edge_index.astype(jnp.int32)
    src_r = ei[0].reshape(NW, NCHUNK, CH)
    dst_r = ei[1].reshape(NW, NCHUNK, CH)

    xp = jnp.pad(x, ((0, NP - N), (0, 0)))

    ones_tab = jnp.ones((NP, HID), jnp.float32)
    degp = _edge_scatter(ones_tab, src_r, dst_r)
    degc = degp[:, :, 0:1]

    dinv, y = _tc0(xp, W0, degc)

    b0r, g0r, be0r = b0.reshape(1, HID), g0.reshape(1, HID), be0.reshape(1, HID)
    b1r, g1r, be1r = b1.reshape(1, HID), g1.reshape(1, HID), be1.reshape(1, HID)
    b2r, g2r, be2r = b2.reshape(1, HID), g2.reshape(1, HID), be2.reshape(1, HID)

    zp = _edge_scatter(y, src_r, dst_r)
    y = _tc_mid(zp, y, dinv, b0r, g0r, be0r, W1)
    zp = _edge_scatter(y, src_r, dst_r)
    y = _tc_mid(zp, y, dinv, b1r, g1r, be1r, W2)
    zp = _edge_scatter(y, src_r, dst_r)
    out = _tc_fin(zp, y, dinv, b2r, g2r, be2r, Wo, bo.reshape(1, 1))
    return out[:N].reshape(N)
